# Initial kernel scaffold; baseline (speedup 1.0000x reference)
import jax
import jax.numpy as jnp
from jax.experimental import pallas as pl


def kernel(x, stem_w, stem_b, sb0_dww, sb0_dwb, sb0_pww, sb0_pwb, sb1_dww, sb1_dwb, sb1_pww, sb1_pwb, sb2_dww, sb2_dwb, sb2_pww, sb2_pwb, sb3_dww, sb3_dwb, sb3_pww, sb3_pwb, tail_dws, tail_dwb, tail_pww, tail_pwb, tail_hw, tail_hb):
    raise NotImplementedError("write your pallas kernel here")


# speedup vs baseline: 4.8118x; 4.8118x over previous
import functools

import jax
import jax.numpy as jnp
from jax.experimental import pallas as pl
from jax.experimental.pallas import tpu as pltpu

_VMEM_LIMIT = 48 * 1024 * 1024


def _prep(x, stride):
    sd, sh, sw = stride
    N, D, H, W, C = x.shape
    osz = lambda n, s: (n - 1) // s + 1
    Do, Ho, Wo = osz(D, sd), osz(H, sh), osz(W, sw)
    Ld, Lh, Lw = Do + 2 // sd, Ho + 2 // sh, Wo + 2 // sw
    xp = jnp.pad(x, ((0, 0),
                     (1, sd * Ld - (D + 1)),
                     (1, sh * Lh - (H + 1)),
                     (1, sw * Lw - (W + 1)),
                     (0, 0)))
    xr = xp.reshape(N, Ld, sd, Lh, sh, Lw, sw, C)
    xr = jnp.transpose(xr, (0, 2, 4, 6, 1, 3, 5, 7))
    return xr.reshape(N, sd * sh * sw, Ld, Lh, Lw, C), (Do, Ho, Wo)


def _taps(stride):
    sd, sh, sw = stride
    out = []
    for kd in range(3):
        for kh in range(3):
            for kw in range(3):
                k = (kd * 3 + kh) * 3 + kw
                phase = ((kd % sd) * sh + kh % sh) * sw + kw % sw
                out.append((k, phase, kd // sd, kh // sh, kw // sw))
    return out


def _stem_kernel(x_ref, w_ref, b_ref, o_ref, *, stride):
    NB, Do, Ho, Wo, Cout = o_ref.shape
    Cin = x_ref.shape[-1]
    rows = NB * Do * Ho * Wo
    acc = jnp.zeros((rows, Cout), jnp.float32)
    for k, phase, od0, oh0, ow0 in _taps(stride):
        tap = x_ref[:, phase, od0:od0 + Do, oh0:oh0 + Ho, ow0:ow0 + Wo, :]
        acc = acc + jnp.dot(tap.reshape(rows, Cin), w_ref[k],
                            preferred_element_type=jnp.float32)
    y = jnp.maximum(acc + b_ref[...], 0.0)
    o_ref[...] = y.reshape(NB, Do, Ho, Wo, Cout)


def _dwpw_kernel(x_ref, dww_ref, dwb_ref, pww_ref, pwb_ref, o_ref, *, stride):
    NB, Do, Ho, Wo, Cout = o_ref.shape
    C = x_ref.shape[-1]
    acc = jnp.zeros((NB, Do, Ho, Wo, C), jnp.float32)
    for k, phase, od0, oh0, ow0 in _taps(stride):
        tap = x_ref[:, phase, od0:od0 + Do, oh0:oh0 + Ho, ow0:ow0 + Wo, :]
        acc = acc + tap * dww_ref[k]
    y = jnp.maximum(acc + dwb_ref[...], 0.0).reshape(NB * Do * Ho * Wo, C)
    z = jnp.dot(y, pww_ref[...], preferred_element_type=jnp.float32)
    z = jnp.maximum(z + pwb_ref[...], 0.0)
    o_ref[...] = z.reshape(NB, Do, Ho, Wo, Cout)


def _final_kernel(x_ref, dww1_ref, dwb1_ref, pww1_ref, pwb1_ref,
                  dww2_ref, dwb2_ref, pww2_ref, pwb2_ref,
                  dww3_ref, dwb3_ref, pww3_ref, pwb3_ref,
                  dws_ref, dwb_ref, pww_ref, pwb_ref, hw_ref, hb_ref,
                  o_ref, *, n_layers, n_classes):
    NB = o_ref.shape[0]
    C1 = x_ref.shape[-1]

    acc = jnp.zeros((NB, 2, 2, 2, C1), jnp.float32)
    for k, phase, od0, oh0, ow0 in _taps((2, 2, 2)):
        tap = x_ref[:, phase, od0:od0 + 2, oh0:oh0 + 2, ow0:ow0 + 2, :]
        acc = acc + tap * dww1_ref[k]
    y = jnp.maximum(acc + dwb1_ref[...], 0.0).reshape(NB * 8, C1)
    h = jnp.dot(y, pww1_ref[...], preferred_element_type=jnp.float32)
    C2 = pww1_ref.shape[-1]
    h = jnp.maximum(h + pwb1_ref[...], 0.0).reshape(NB, 2, 2, 2, C2)

    zw = jnp.zeros((NB, 2, 2, 1, C2), jnp.float32)
    hp = jnp.concatenate([zw, h, zw], axis=3)
    zh = jnp.zeros((NB, 2, 1, 4, C2), jnp.float32)
    hp = jnp.concatenate([zh, hp, zh], axis=2)
    zd = jnp.zeros((NB, 1, 4, 4, C2), jnp.float32)
    hp = jnp.concatenate([zd, hp, zd], axis=1)
    acc2 = jnp.zeros((NB, 2, 2, 2, C2), jnp.float32)
    for kd in range(3):
        for kh in range(3):
            for kw in range(3):
                k = (kd * 3 + kh) * 3 + kw
                tap = hp[:, kd:kd + 2, kh:kh + 2, kw:kw + 2, :]
                acc2 = acc2 + tap * dww2_ref[k]
    y2 = jnp.maximum(acc2 + dwb2_ref[...], 0.0).reshape(NB * 8, C2)
    h2 = jnp.dot(y2, pww2_ref[...], preferred_element_type=jnp.float32)
    C3 = pww2_ref.shape[-1]
    h2 = jnp.maximum(h2 + pwb2_ref[...], 0.0).reshape(NB, 2, 2, 2, C3)

    acc3 = jnp.zeros((NB, C3), jnp.float32)
    for kd in (1, 2):
        for kh in (1, 2):
            for kw in (1, 2):
                k = (kd * 3 + kh) * 3 + kw
                acc3 = acc3 + h2[:, kd - 1, kh - 1, kw - 1, :] * dww3_ref[k]
    y3 = jnp.maximum(acc3 + dwb3_ref[...], 0.0)
    z = jnp.dot(y3, pww3_ref[...], preferred_element_type=jnp.float32)
    z = jnp.maximum(z + pwb3_ref[...], 0.0)

    Cp = pww_ref.shape[-1]
    ht = jnp.concatenate(
        [z, jnp.zeros((NB, Cp - z.shape[-1]), jnp.float32)], axis=1)
    for l in range(n_layers):
        yt = jnp.maximum(ht * dws_ref[l] + dwb_ref[l], 0.0)
        ht = jnp.maximum(
            jnp.dot(yt, pww_ref[l], preferred_element_type=jnp.float32)
            + pwb_ref[l], 0.0)
    logits = (jnp.dot(ht, hw_ref[...], preferred_element_type=jnp.float32)
              + hb_ref[...])
    lane = jax.lax.broadcasted_iota(jnp.int32, logits.shape, dimension=1)
    logits = jnp.where(lane < n_classes, logits, jnp.float32(-1e30))
    m = jnp.max(logits, axis=-1, keepdims=True)
    e = jnp.exp(logits - m)
    o_ref[...] = e / jnp.sum(e, axis=-1, keepdims=True)


def _full(shape):
    nd = len(shape)
    return pl.BlockSpec(shape, lambda n, _nd=nd: (0,) * _nd)


def kernel(x, stem_w, stem_b, sb0_dww, sb0_dwb, sb0_pww, sb0_pwb,
           sb1_dww, sb1_dwb, sb1_pww, sb1_pwb, sb2_dww, sb2_dwb, sb2_pww,
           sb2_pwb, sb3_dww, sb3_dwb, sb3_pww, sb3_pwb,
           tail_dws, tail_dwb, tail_pww, tail_pwb, tail_hw, tail_hb):
    N = x.shape[0]
    x = jnp.transpose(x, (0, 2, 3, 4, 1))
    cin_pad = stem_w.shape[1]
    x = jnp.pad(x, ((0, 0), (0, 0), (0, 0), (0, 0),
                    (0, cin_pad - x.shape[-1])))

    nb0 = 4
    xp, (Do, Ho, Wo) = _prep(x, (1, 2, 2))
    _, S, Ld, Lh, Lw, Ci = xp.shape
    Cout = stem_w.shape[-1]
    y = pl.pallas_call(
        functools.partial(_stem_kernel, stride=(1, 2, 2)),
        out_shape=jax.ShapeDtypeStruct((N, Do, Ho, Wo, Cout), jnp.float32),
        grid=(N // nb0,),
        in_specs=[
            pl.BlockSpec((nb0, S, Ld, Lh, Lw, Ci),
                         lambda n: (n, 0, 0, 0, 0, 0)),
            _full(stem_w.shape), _full(stem_b.shape),
        ],
        out_specs=pl.BlockSpec((nb0, Do, Ho, Wo, Cout),
                               lambda n: (n, 0, 0, 0, 0)),
        compiler_params=pltpu.CompilerParams(
            dimension_semantics=("parallel",),
            vmem_limit_bytes=_VMEM_LIMIT),
    )(xp, stem_w, stem_b)

    nb1 = 8
    xp, (Do, Ho, Wo) = _prep(y, (2, 2, 2))
    _, S, Ld, Lh, Lw, Ci = xp.shape
    Cout = sb0_pww.shape[-1]
    y = pl.pallas_call(
        functools.partial(_dwpw_kernel, stride=(2, 2, 2)),
        out_shape=jax.ShapeDtypeStruct((N, Do, Ho, Wo, Cout), jnp.float32),
        grid=(N // nb1,),
        in_specs=[
            pl.BlockSpec((nb1, S, Ld, Lh, Lw, Ci),
                         lambda n: (n, 0, 0, 0, 0, 0)),
            _full(sb0_dww.shape), _full(sb0_dwb.shape),
            _full(sb0_pww.shape), _full(sb0_pwb.shape),
        ],
        out_specs=pl.BlockSpec((nb1, Do, Ho, Wo, Cout),
                               lambda n: (n, 0, 0, 0, 0)),
        compiler_params=pltpu.CompilerParams(
            dimension_semantics=("parallel",),
            vmem_limit_bytes=_VMEM_LIMIT),
    )(xp, sb0_dww, sb0_dwb, sb0_pww, sb0_pwb)

    nb2 = 32
    n_layers = tail_dws.shape[0]
    n_classes = 10
    xp, _ = _prep(y, (2, 2, 2))
    _, S, Ld, Lh, Lw, Ci = xp.shape
    Kp = tail_hw.shape[-1]
    wts = [sb1_dww, sb1_dwb, sb1_pww, sb1_pwb,
           sb2_dww, sb2_dwb, sb2_pww, sb2_pwb,
           sb3_dww, sb3_dwb, sb3_pww, sb3_pwb,
           tail_dws, tail_dwb, tail_pww, tail_pwb, tail_hw, tail_hb]
    probs = pl.pallas_call(
        functools.partial(_final_kernel, n_layers=n_layers,
                          n_classes=n_classes),
        out_shape=jax.ShapeDtypeStruct((N, Kp), jnp.float32),
        grid=(N // nb2,),
        in_specs=[pl.BlockSpec((nb2, S, Ld, Lh, Lw, Ci),
                               lambda n: (n, 0, 0, 0, 0, 0))]
                 + [_full(w.shape) for w in wts],
        out_specs=pl.BlockSpec((nb2, Kp), lambda n: (n, 0)),
        compiler_params=pltpu.CompilerParams(
            dimension_semantics=("parallel",),
            vmem_limit_bytes=_VMEM_LIMIT),
    )(xp, *wts)
    return probs[:, :n_classes]
